# baseline (speedup 1.0000x reference)
import functools

import jax
import jax.numpy as jnp
from jax.experimental import pallas as pl

_BS, _NN, _D = 2, 512, 128
_NUM_REL = 2

_CONTRACT0 = (((0,), (0,)), ((), ()))


def _split(v):
    vh = v.astype(jnp.bfloat16)
    vl = (v - vh.astype(jnp.float32)).astype(jnp.bfloat16)
    return vh, vl


def _mm3(u, wh, wl):
    uh, ul = _split(u)
    return (jnp.dot(uh, wh, preferred_element_type=jnp.float32)
            + jnp.dot(uh, wl, preferred_element_type=jnp.float32)
            + jnp.dot(ul, wh, preferred_element_type=jnp.float32))


def _aggc(a, xh, xl, ones8):
    s = jax.lax.dot_general(a, xh, _CONTRACT0,
                            preferred_element_type=jnp.float32)
    s = s + jax.lax.dot_general(a, xl, _CONTRACT0,
                                preferred_element_type=jnp.float32)
    cnt = jax.lax.dot_general(a, ones8, _CONTRACT0,
                              preferred_element_type=jnp.float32)[:, :1]
    return s, cnt


def _rgcn_kernel(adj_ref, x_ref, wrel0_ref, wroot0_ref, b0_ref,
                 wrel1_ref, wroot1_ref, b1_ref, out_ref):
    ones8 = jnp.ones((_NN, 8), dtype=jnp.bfloat16)

    ws = []
    for wrel_ref, wroot_ref, b_ref in ((wrel0_ref, wroot0_ref, b0_ref),
                                       (wrel1_ref, wroot1_ref, b1_ref)):
        ws.append((_split(wroot_ref[...]), _split(wrel_ref[0]),
                   _split(wrel_ref[1]), b_ref[...]))

    a1 = []
    a0 = []
    inv = [[None, None] for _ in range(_BS)]
    for b in range(_BS):
        aug = adj_ref[0, b]
        pun = adj_ref[1, b]
        a1b = aug.astype(jnp.float32)
        a0b = pun.astype(jnp.float32) * (1.0 - a1b)
        a1.append(a1b.astype(jnp.bfloat16))
        a0.append(a0b.astype(jnp.bfloat16))

    x = x_ref[...].reshape(_BS * _NN, _D)
    for (wrh, wrl), (w0h, w0l), (w1h, w1l), bias in ws:
        xh, xl = _split(x)
        parts = []
        for b in range(_BS):
            xhb = xh[b * _NN:(b + 1) * _NN]
            xlb = xl[b * _NN:(b + 1) * _NN]
            s0, c0 = _aggc(a0[b], xhb, xlb, ones8)
            s1, c1 = _aggc(a1[b], xhb, xlb, ones8)
            parts.append((s0 / jnp.maximum(c0, 1.0),
                          s1 / jnp.maximum(c1, 1.0)))
        u0 = jnp.concatenate([p[0] for p in parts], axis=0)
        u1 = jnp.concatenate([p[1] for p in parts], axis=0)
        h = _mm3(x, wrh, wrl) + bias
        h = h + _mm3(u0, w0h, w0l)
        h = h + _mm3(u1, w1h, w1l)
        x = jnp.where(h > 0, h, jnp.exp(jnp.minimum(h, 0.0)) - 1.0)
    out_ref[...] = x.reshape(_BS, _NN, _D)


@functools.partial(jax.jit, static_argnames=())
def _run(adj, x, wrel0, wroot0, b0, wrel1, wroot1, b1):
    return pl.pallas_call(
        _rgcn_kernel,
        out_shape=jax.ShapeDtypeStruct((_BS, _NN, _D), jnp.float32),
    )(adj, x, wrel0, wroot0, b0, wrel1, wroot1, b1)


def kernel(feature_list, adj_list, aug_pun_adj, pooled_output, p_nodes_mask,
           o_nodes_mask, W_rel0, W_root0, bias0, W_rel1, W_root1, bias1):
    x = feature_list[0]
    adj = aug_pun_adj.astype(jnp.int32)
    out = _run(adj, x, W_rel0, W_root0, bias0.reshape(1, _D),
               W_rel1, W_root1, bias1.reshape(1, _D))
    return out


# speedup vs baseline: 1.0478x; 1.0478x over previous
import functools

import jax
import jax.numpy as jnp
from jax.experimental import pallas as pl

_BS, _NN, _D = 2, 512, 128
_NUM_REL = 2

_CONTRACT0 = (((0,), (0,)), ((), ()))


def _split(v):
    vh = v.astype(jnp.bfloat16)
    vl = (v - vh.astype(jnp.float32)).astype(jnp.bfloat16)
    return vh, vl


def _mm3(u, wh, wl):
    uh, ul = _split(u)
    return (jnp.dot(uh, wh, preferred_element_type=jnp.float32)
            + jnp.dot(uh, wl, preferred_element_type=jnp.float32)
            + jnp.dot(ul, wh, preferred_element_type=jnp.float32))


def _agg(a, xh, xl):
    s = jax.lax.dot_general(a, xh, _CONTRACT0,
                            preferred_element_type=jnp.float32)
    return s + jax.lax.dot_general(a, xl, _CONTRACT0,
                                   preferred_element_type=jnp.float32)


def _rgcn_kernel(adj_ref, x_ref, wrel0_ref, wroot0_ref, b0_ref,
                 wrel1_ref, wroot1_ref, b1_ref, out_ref):
    aug = adj_ref[0, 0]
    pun = adj_ref[1, 0]
    a1f = aug.astype(jnp.float32)
    a0f = pun.astype(jnp.float32) * (1.0 - a1f)
    a1 = a1f.astype(jnp.bfloat16)
    a0 = a0f.astype(jnp.bfloat16)

    ones8 = jnp.ones((_NN, 8), dtype=jnp.bfloat16)
    cnt0 = jax.lax.dot_general(a0, ones8, _CONTRACT0,
                               preferred_element_type=jnp.float32)[:, :1]
    cnt1 = jax.lax.dot_general(a1, ones8, _CONTRACT0,
                               preferred_element_type=jnp.float32)[:, :1]
    inv0 = 1.0 / jnp.maximum(cnt0, 1.0)
    inv1 = 1.0 / jnp.maximum(cnt1, 1.0)

    x = x_ref[0]
    for wrel_ref, wroot_ref, b_ref in ((wrel0_ref, wroot0_ref, b0_ref),
                                       (wrel1_ref, wroot1_ref, b1_ref)):
        wrh, wrl = _split(wroot_ref[...])
        w0h, w0l = _split(wrel_ref[0])
        w1h, w1l = _split(wrel_ref[1])
        xh, xl = _split(x)
        h = _mm3(x, wrh, wrl) + b_ref[...]
        h = h + _mm3(_agg(a0, xh, xl) * inv0, w0h, w0l)
        h = h + _mm3(_agg(a1, xh, xl) * inv1, w1h, w1l)
        x = jnp.where(h > 0, h, jnp.exp(jnp.minimum(h, 0.0)) - 1.0)
    out_ref[0] = x


@functools.partial(jax.jit, static_argnames=())
def _run(adj, x, wrel0, wroot0, b0, wrel1, wroot1, b1):
    return pl.pallas_call(
        _rgcn_kernel,
        grid=(_BS,),
        in_specs=[
            pl.BlockSpec((2, 1, _NN, _NN), lambda b: (0, b, 0, 0)),
            pl.BlockSpec((1, _NN, _D), lambda b: (b, 0, 0)),
            pl.BlockSpec((_NUM_REL, _D, _D), lambda b: (0, 0, 0)),
            pl.BlockSpec((_D, _D), lambda b: (0, 0)),
            pl.BlockSpec((1, _D), lambda b: (0, 0)),
            pl.BlockSpec((_NUM_REL, _D, _D), lambda b: (0, 0, 0)),
            pl.BlockSpec((_D, _D), lambda b: (0, 0)),
            pl.BlockSpec((1, _D), lambda b: (0, 0)),
        ],
        out_specs=pl.BlockSpec((1, _NN, _D), lambda b: (b, 0, 0)),
        out_shape=jax.ShapeDtypeStruct((_BS, _NN, _D), jnp.float32),
    )(adj, x, wrel0, wroot0, b0, wrel1, wroot1, b1)


def kernel(feature_list, adj_list, aug_pun_adj, pooled_output, p_nodes_mask,
           o_nodes_mask, W_rel0, W_root0, bias0, W_rel1, W_root1, bias1):
    x = feature_list[0]
    adj = aug_pun_adj.astype(jnp.int32)
    out = _run(adj, x, W_rel0, W_root0, bias0.reshape(1, _D),
               W_rel1, W_root1, bias1.reshape(1, _D))
    return out


# speedup vs baseline: 1.4732x; 1.4060x over previous
import functools

import jax
import jax.numpy as jnp
from jax.experimental import pallas as pl

_BS, _NN, _D = 2, 512, 128
_NUM_REL = 2

_CONTRACT0 = (((0,), (0,)), ((), ()))


def _split(v):
    vh = v.astype(jnp.bfloat16)
    vl = (v - vh.astype(jnp.float32)).astype(jnp.bfloat16)
    return vh, vl


def _mm3(u, wh, wl):
    uh, ul = _split(u)
    return (jnp.dot(uh, wh, preferred_element_type=jnp.float32)
            + jnp.dot(uh, wl, preferred_element_type=jnp.float32)
            + jnp.dot(ul, wh, preferred_element_type=jnp.float32))


def _agg(a, xh):
    return jax.lax.dot_general(a, xh, _CONTRACT0,
                               preferred_element_type=jnp.float32)


def _rgcn_kernel(adj_ref, x_ref, wrel0_ref, wroot0_ref, b0_ref,
                 wrel1_ref, wroot1_ref, b1_ref, out_ref):
    aug = adj_ref[0, 0]
    pun = adj_ref[1, 0]
    m1 = aug == 1
    m0 = (pun == 1) & (aug != 1)
    a1 = m1.astype(jnp.bfloat16)
    a0 = m0.astype(jnp.bfloat16)

    inv0 = 1.0 / jnp.maximum(jnp.sum(m0.astype(jnp.float32), axis=0), 1.0)
    inv1 = 1.0 / jnp.maximum(jnp.sum(m1.astype(jnp.float32), axis=0), 1.0)

    x = x_ref[0]
    for wrel_ref, wroot_ref, b_ref in ((wrel0_ref, wroot0_ref, b0_ref),
                                       (wrel1_ref, wroot1_ref, b1_ref)):
        wrh, wrl = _split(wroot_ref[...])
        w0h, w0l = _split(wrel_ref[0])
        w1h, w1l = _split(wrel_ref[1])
        xh = x.astype(jnp.bfloat16)
        h = _mm3(x, wrh, wrl) + b_ref[...]
        h = h + _mm3(_agg(a0, xh) * inv0[:, None], w0h, w0l)
        h = h + _mm3(_agg(a1, xh) * inv1[:, None], w1h, w1l)
        x = jnp.where(h > 0, h, jnp.exp(jnp.minimum(h, 0.0)) - 1.0)
    out_ref[0] = x


@functools.partial(jax.jit, static_argnames=())
def _run(adj, x, wrel0, wroot0, b0, wrel1, wroot1, b1):
    return pl.pallas_call(
        _rgcn_kernel,
        grid=(_BS,),
        in_specs=[
            pl.BlockSpec((2, 1, _NN, _NN), lambda b: (0, b, 0, 0)),
            pl.BlockSpec((1, _NN, _D), lambda b: (b, 0, 0)),
            pl.BlockSpec((_NUM_REL, _D, _D), lambda b: (0, 0, 0)),
            pl.BlockSpec((_D, _D), lambda b: (0, 0)),
            pl.BlockSpec((1, _D), lambda b: (0, 0)),
            pl.BlockSpec((_NUM_REL, _D, _D), lambda b: (0, 0, 0)),
            pl.BlockSpec((_D, _D), lambda b: (0, 0)),
            pl.BlockSpec((1, _D), lambda b: (0, 0)),
        ],
        out_specs=pl.BlockSpec((1, _NN, _D), lambda b: (b, 0, 0)),
        out_shape=jax.ShapeDtypeStruct((_BS, _NN, _D), jnp.float32),
    )(adj, x, wrel0, wroot0, b0, wrel1, wroot1, b1)


def kernel(feature_list, adj_list, aug_pun_adj, pooled_output, p_nodes_mask,
           o_nodes_mask, W_rel0, W_root0, bias0, W_rel1, W_root1, bias1):
    x = feature_list[0]
    adj = aug_pun_adj.astype(jnp.int32)
    out = _run(adj, x, W_rel0, W_root0, bias0.reshape(1, _D),
               W_rel1, W_root1, bias1.reshape(1, _D))
    return out


# speedup vs baseline: 1.5829x; 1.0745x over previous
import functools

import jax
import jax.numpy as jnp
from jax.experimental import pallas as pl

_BS, _NN, _D = 2, 512, 128
_NUM_REL = 2

_CONTRACT0 = (((0,), (0,)), ((), ()))


def _split(v):
    vh = v.astype(jnp.bfloat16)
    vl = (v - vh.astype(jnp.float32)).astype(jnp.bfloat16)
    return vh, vl


def _mm3(xh, xl, wh, wl):
    return (jnp.dot(xh, wh, preferred_element_type=jnp.float32)
            + jnp.dot(xh, wl, preferred_element_type=jnp.float32)
            + jnp.dot(xl, wh, preferred_element_type=jnp.float32))


def _mm2(xh, xl, wh, wl):
    return (jnp.dot(xh, wh, preferred_element_type=jnp.float32)
            + jnp.dot(xh, wl, preferred_element_type=jnp.float32))


def _agg(a, yh):
    return jax.lax.dot_general(a, yh, _CONTRACT0,
                               preferred_element_type=jnp.float32)


def _rgcn_kernel(adj_ref, x_ref, wrel0_ref, wroot0_ref, b0_ref,
                 wrel1_ref, wroot1_ref, b1_ref, out_ref):
    aug = adj_ref[0, 0]
    pun = adj_ref[1, 0]
    m1 = aug == 1
    m0 = (pun == 1) & (aug != 1)
    a1 = m1.astype(jnp.bfloat16)
    a0 = m0.astype(jnp.bfloat16)

    inv0 = 1.0 / jnp.maximum(jnp.sum(m0.astype(jnp.float32), axis=0), 1.0)
    inv1 = 1.0 / jnp.maximum(jnp.sum(m1.astype(jnp.float32), axis=0), 1.0)

    x = x_ref[0]
    for wrel_ref, wroot_ref, b_ref in ((wrel0_ref, wroot0_ref, b0_ref),
                                       (wrel1_ref, wroot1_ref, b1_ref)):
        wrh, wrl = _split(wroot_ref[...])
        w0h, w0l = _split(wrel_ref[0])
        w1h, w1l = _split(wrel_ref[1])
        xh, xl = _split(x)
        hroot = _mm3(xh, xl, wrh, wrl) + b_ref[...]
        y0h = _mm2(xh, xl, w0h, w0l).astype(jnp.bfloat16)
        y1h = _mm2(xh, xl, w1h, w1l).astype(jnp.bfloat16)
        h = (hroot + _agg(a0, y0h) * inv0[:, None]
             + _agg(a1, y1h) * inv1[:, None])
        x = jnp.where(h > 0, h, jnp.exp(jnp.minimum(h, 0.0)) - 1.0)
    out_ref[0] = x


@functools.partial(jax.jit, static_argnames=())
def _run(adj, x, wrel0, wroot0, b0, wrel1, wroot1, b1):
    return pl.pallas_call(
        _rgcn_kernel,
        grid=(_BS,),
        in_specs=[
            pl.BlockSpec((2, 1, _NN, _NN), lambda b: (0, b, 0, 0)),
            pl.BlockSpec((1, _NN, _D), lambda b: (b, 0, 0)),
            pl.BlockSpec((_NUM_REL, _D, _D), lambda b: (0, 0, 0)),
            pl.BlockSpec((_D, _D), lambda b: (0, 0)),
            pl.BlockSpec((1, _D), lambda b: (0, 0)),
            pl.BlockSpec((_NUM_REL, _D, _D), lambda b: (0, 0, 0)),
            pl.BlockSpec((_D, _D), lambda b: (0, 0)),
            pl.BlockSpec((1, _D), lambda b: (0, 0)),
        ],
        out_specs=pl.BlockSpec((1, _NN, _D), lambda b: (b, 0, 0)),
        out_shape=jax.ShapeDtypeStruct((_BS, _NN, _D), jnp.float32),
    )(adj, x, wrel0, wroot0, b0, wrel1, wroot1, b1)


def kernel(feature_list, adj_list, aug_pun_adj, pooled_output, p_nodes_mask,
           o_nodes_mask, W_rel0, W_root0, bias0, W_rel1, W_root1, bias1):
    x = feature_list[0]
    adj = aug_pun_adj.astype(jnp.int32)
    out = _run(adj, x, W_rel0, W_root0, bias0.reshape(1, _D),
               W_rel1, W_root1, bias1.reshape(1, _D))
    return out
